# baseline (speedup 1.0000x reference)
import functools

import jax
import jax.numpy as jnp
from jax import lax
from jax.experimental import pallas as pl
from jax.experimental.pallas import tpu as pltpu
from jax.experimental.pallas import tpu_sc as plsc

BATCH = 4096
HIST = 200
EMBED_DIM = 64
PADDED_DIM = 128

_info = plsc.get_sparse_core_info()
NUM_CORES = _info.num_cores
NUM_SUBCORES = _info.num_subcores
NUM_WORKERS = NUM_CORES * NUM_SUBCORES
B_PER_W = BATCH // NUM_WORKERS


def _gather_body(x_hbm, table_hbm, out_hbm, idx_v, rows_a, rows_b,
                 gs_a, gs_b, ws_a, ws_b):
    wid = lax.axis_index("s") * NUM_CORES + lax.axis_index("c")
    b0 = wid * B_PER_W
    pltpu.sync_copy(x_hbm.at[pl.ds(b0 * HIST, B_PER_W * HIST)], idx_v)

    def gather_start(i, buf, sem):
        pltpu.async_copy(table_hbm.at[idx_v.at[pl.ds(i * HIST, HIST)]],
                         buf, sem)

    def gather_wait(buf, sem):
        pltpu.make_async_copy(table_hbm.at[idx_v.at[pl.ds(0, HIST)]],
                              buf, sem).wait()

    def write_start(i, buf, sem):
        pltpu.async_copy(buf, out_hbm.at[b0 + i, :, pl.ds(0, EMBED_DIM)],
                         sem)

    def write_wait(buf, sem):
        pltpu.make_async_copy(buf, out_hbm.at[b0, :, pl.ds(0, EMBED_DIM)],
                              sem).wait()

    gather_start(0, rows_a, gs_a)
    gather_start(1, rows_b, gs_b)

    def step(k, carry):
        i = 2 * k
        gather_wait(rows_a, gs_a)
        write_start(i, rows_a, ws_a)
        gather_wait(rows_b, gs_b)
        write_start(i + 1, rows_b, ws_b)
        write_wait(rows_a, ws_a)
        gather_start(i + 2, rows_a, gs_a)
        write_wait(rows_b, ws_b)
        gather_start(i + 3, rows_b, gs_b)
        return carry

    lax.fori_loop(0, (B_PER_W - 2) // 2, step, 0)

    gather_wait(rows_a, gs_a)
    write_start(B_PER_W - 2, rows_a, ws_a)
    gather_wait(rows_b, gs_b)
    write_start(B_PER_W - 1, rows_b, ws_b)
    write_wait(rows_a, ws_a)
    write_wait(rows_b, ws_b)


def _gather(x_flat, table_padded):
    mesh = plsc.VectorSubcoreMesh(core_axis_name="c", subcore_axis_name="s")
    kfn = functools.partial(
        pl.kernel,
        mesh=mesh,
        out_type=jax.ShapeDtypeStruct((BATCH, HIST, PADDED_DIM), jnp.float32),
        scratch_types=[
            pltpu.VMEM((B_PER_W * HIST,), jnp.int32),
            pltpu.VMEM((HIST, EMBED_DIM), jnp.float32),
            pltpu.VMEM((HIST, EMBED_DIM), jnp.float32),
            pltpu.SemaphoreType.DMA,
            pltpu.SemaphoreType.DMA,
            pltpu.SemaphoreType.DMA,
            pltpu.SemaphoreType.DMA,
        ],
        compiler_params=pltpu.CompilerParams(use_tc_tiling_on_sc=False),
    )(_gather_body)
    return kfn(x_flat, table_padded)


TBLK = 512


def _transpose_body(t_ref, o_ref):
    r = lax.broadcasted_iota(jnp.int32, (EMBED_DIM, EMBED_DIM), 0)
    c = lax.broadcasted_iota(jnp.int32, (EMBED_DIM, EMBED_DIM), 1)
    eye = jnp.where(r == c, 1.0, 0.0).astype(jnp.float32)
    bt = lax.dot_general(t_ref[...], eye, (((0,), (0,)), ((), ())),
                         preferred_element_type=jnp.float32)
    o_ref[...] = jnp.concatenate([bt[:TBLK // 2], bt[TBLK // 2:]], axis=1)


def _pack_table(table_t):
    vocab = table_t.shape[1]
    nblk = pl.cdiv(vocab, TBLK)
    return pl.pallas_call(
        _transpose_body,
        grid=(nblk,),
        in_specs=[pl.BlockSpec((EMBED_DIM, TBLK), lambda j: (0, j))],
        out_specs=pl.BlockSpec((TBLK // 2, 2 * EMBED_DIM), lambda j: (j, 0)),
        out_shape=jax.ShapeDtypeStruct((nblk * TBLK // 2, 2 * EMBED_DIM),
                                       jnp.float32),
    )(table_t)


def kernel(x, table):
    xi = x.reshape(-1).astype(jnp.int32)
    half = TBLK // 2
    x_flat = ((xi & ~(TBLK - 1)) + 2 * (xi % half) + (xi % TBLK) // half)
    packed = _pack_table(table.T)
    t_lin = packed.reshape(-1, EMBED_DIM)
    return _gather(x_flat, t_lin)[..., :EMBED_DIM]


# speedup vs baseline: 2.3626x; 2.3626x over previous
import functools

import jax
import jax.numpy as jnp
from jax import lax
from jax.experimental import pallas as pl
from jax.experimental.pallas import tpu as pltpu
from jax.experimental.pallas import tpu_sc as plsc

BATCH = 4096
HIST = 200
EMBED_DIM = 64
PADDED_DIM = 128

_info = plsc.get_sparse_core_info()
NUM_CORES = _info.num_cores
NUM_SUBCORES = _info.num_subcores
NUM_WORKERS = NUM_CORES * NUM_SUBCORES
B_PER_W = BATCH // NUM_WORKERS


def _gather_body(x_hbm, table_hbm, out_hbm, idx_v, rows_a, rows_b,
                 gs_a, gs_b, ws_a, ws_b):
    wid = lax.axis_index("s") * NUM_CORES + lax.axis_index("c")
    b0 = wid * B_PER_W
    pltpu.sync_copy(x_hbm.at[pl.ds(b0 * HIST, B_PER_W * HIST)], idx_v)

    def gather_start(i, buf, sem):
        pltpu.async_copy(table_hbm.at[idx_v.at[pl.ds(i * HIST, HIST)]],
                         buf, sem)

    def gather_wait(buf, sem):
        pltpu.make_async_copy(table_hbm.at[idx_v.at[pl.ds(0, HIST)]],
                              buf, sem).wait()

    def write_start(i, buf, sem):
        pltpu.async_copy(buf, out_hbm.at[b0 + i, :, pl.ds(0, EMBED_DIM)],
                         sem)

    def write_wait(buf, sem):
        pltpu.make_async_copy(buf, out_hbm.at[b0, :, pl.ds(0, EMBED_DIM)],
                              sem).wait()

    gather_start(0, rows_a, gs_a)
    gather_start(1, rows_b, gs_b)

    def step(k, carry):
        i = 2 * k
        gather_wait(rows_a, gs_a)
        write_start(i, rows_a, ws_a)
        gather_wait(rows_b, gs_b)
        write_start(i + 1, rows_b, ws_b)
        write_wait(rows_a, ws_a)
        gather_start(i + 2, rows_a, gs_a)
        write_wait(rows_b, ws_b)
        gather_start(i + 3, rows_b, gs_b)
        return carry

    lax.fori_loop(0, (B_PER_W - 2) // 2, step, 0)

    gather_wait(rows_a, gs_a)
    write_start(B_PER_W - 2, rows_a, ws_a)
    gather_wait(rows_b, gs_b)
    write_start(B_PER_W - 1, rows_b, ws_b)
    write_wait(rows_a, ws_a)
    write_wait(rows_b, ws_b)


def _gather(x_flat, table_padded):
    mesh = plsc.VectorSubcoreMesh(core_axis_name="c", subcore_axis_name="s")
    kfn = functools.partial(
        pl.kernel,
        mesh=mesh,
        out_type=jax.ShapeDtypeStruct((BATCH, HIST, PADDED_DIM), jnp.float32),
        scratch_types=[
            pltpu.VMEM((B_PER_W * HIST,), jnp.int32),
            pltpu.VMEM((HIST, EMBED_DIM), jnp.float32),
            pltpu.VMEM((HIST, EMBED_DIM), jnp.float32),
            pltpu.SemaphoreType.DMA,
            pltpu.SemaphoreType.DMA,
            pltpu.SemaphoreType.DMA,
            pltpu.SemaphoreType.DMA,
        ],
        compiler_params=pltpu.CompilerParams(use_tc_tiling_on_sc=False),
    )(_gather_body)
    return kfn(x_flat, table_padded)


TBLK = 4096


def _transpose_body(t_ref, o_ref):
    bt = t_ref[...].T
    o_ref[:, :EMBED_DIM] = bt[:TBLK // 2]
    o_ref[:, EMBED_DIM:] = bt[TBLK // 2:]


def _pack_table(table_t):
    vocab = table_t.shape[1]
    nblk = pl.cdiv(vocab, TBLK)
    return pl.pallas_call(
        _transpose_body,
        grid=(nblk,),
        in_specs=[pl.BlockSpec((EMBED_DIM, TBLK), lambda j: (0, j))],
        out_specs=pl.BlockSpec((TBLK // 2, 2 * EMBED_DIM), lambda j: (j, 0)),
        out_shape=jax.ShapeDtypeStruct((nblk * TBLK // 2, 2 * EMBED_DIM),
                                       jnp.float32),
    )(table_t)


def kernel(x, table):
    xi = x.reshape(-1).astype(jnp.int32)
    half = TBLK // 2
    x_flat = ((xi & ~(TBLK - 1)) + 2 * (xi % half) + (xi % TBLK) // half)
    packed = _pack_table(table.T)
    t_lin = packed.reshape(-1, EMBED_DIM)
    return _gather(x_flat, t_lin)[..., :EMBED_DIM]
